# Initial kernel scaffold; baseline (speedup 1.0000x reference)
import jax
import jax.numpy as jnp
from jax.experimental import pallas as pl


def kernel(x, pos, w_start, b_start, w_key, b_key, w_query, b_query, w_value, b_value, w_pos1, b_pos1, g_pos, be_pos, w_pos2, b_pos2, w_attn1, b_attn1, g_attn, be_attn, w_attn2, b_attn2, w_end, b_end):
    raise NotImplementedError("write your pallas kernel here")


# speedup vs baseline: 93.7483x; 93.7483x over previous
import functools

import jax
import jax.numpy as jnp
from jax import lax
from jax.experimental import pallas as pl
from jax.experimental.pallas import tpu as pltpu
from jax.experimental.pallas import tpu_sc as plsc

N = 10000
K = 16
NPAD = 10112
DIM = 64
CIN = 128
HID = 256
QTILE = 128
NT = 200
ST = NT * K
NSAMP = N * K
NSAMP_PAD = 163840
BIGF = 3.0e38


def _qkv_body(x_ref, ws_ref, wqkv_ref, o_ref):
    xf = jnp.dot(x_ref[...], ws_ref[...], preferred_element_type=jnp.float32)
    o_ref[...] = jnp.dot(xf, wqkv_ref[...], preferred_element_type=jnp.float32)


def _qkv(xT, wsT, wqkvT):
    tn = 1000
    return pl.pallas_call(
        _qkv_body,
        grid=(N // tn,),
        in_specs=[
            pl.BlockSpec((tn, CIN), lambda i: (i, 0)),
            pl.BlockSpec((CIN, DIM), lambda i: (0, 0)),
            pl.BlockSpec((DIM, 3 * DIM), lambda i: (0, 0)),
        ],
        out_specs=pl.BlockSpec((tn, 3 * DIM), lambda i: (i, 0)),
        out_shape=jax.ShapeDtypeStruct((N, 3 * DIM), jnp.float32),
    )(xT, wsT, wqkvT)


def _knn_body(q_ref, p_ref, o_ref):
    Q = q_ref[...]
    P = p_ref[...]
    S = jnp.sum(P * P, axis=0, keepdims=True)
    D = S - 2.0 * jnp.dot(Q, P, preferred_element_type=jnp.float32)
    colid = lax.broadcasted_iota(jnp.int32, (QTILE, NPAD), 1)
    D = jnp.where(colid >= N, BIGF, D)
    cols = []
    for _ in range(K):
        m = jnp.min(D, axis=1, keepdims=True)
        cand = jnp.where(D == m, colid, NPAD)
        sel = jnp.min(cand, axis=1, keepdims=True)
        cols.append(sel)
        D = jnp.where(colid == sel, BIGF, D)
    o_ref[...] = jnp.concatenate(cols, axis=1)


def _knn(qpos, pos8):
    return pl.pallas_call(
        _knn_body,
        grid=(NPAD // QTILE,),
        in_specs=[
            pl.BlockSpec((QTILE, 8), lambda i: (i, 0)),
            pl.BlockSpec((8, NPAD), lambda i: (0, 0)),
        ],
        out_specs=pl.BlockSpec((QTILE, K), lambda i: (i, 0)),
        out_shape=jax.ShapeDtypeStruct((NPAD, K), jnp.int32),
    )(qpos, pos8)


_SC_CH = 128
_SC_USE = True


def _sc_gather(keyT, posT16, idx2d):
    info = plsc.get_sparse_core_info()
    nw = info.num_cores * info.num_subcores
    rows_per_w = idx2d.shape[0] // nw
    mesh = plsc.VectorSubcoreMesh(core_axis_name="c", subcore_axis_name="s")

    @functools.partial(
        pl.kernel,
        mesh=mesh,
        compiler_params=pltpu.CompilerParams(use_tc_tiling_on_sc=False),
        out_type=[
            jax.ShapeDtypeStruct((NSAMP_PAD, DIM), jnp.float32),
            jax.ShapeDtypeStruct((NSAMP_PAD, 16), jnp.float32),
        ],
        scratch_types=[
            pltpu.VMEM((_SC_CH,), jnp.int32),
            pltpu.VMEM((_SC_CH, DIM), jnp.float32),
            pltpu.VMEM((_SC_CH, 16), jnp.float32),
            pltpu.SemaphoreType.DMA,
            pltpu.SemaphoreType.DMA,
        ],
    )
    def gk(key_hbm, pos_hbm, idx_hbm, outk_hbm, outp_hbm,
           idx_v, krows, prows, sem1, sem2):
        wid = lax.axis_index("s") * info.num_cores + lax.axis_index("c")
        row0 = wid * rows_per_w

        def body(i, _):
            row = row0 + i
            off = row * _SC_CH
            pltpu.sync_copy(idx_hbm.at[row], idx_v)
            a = pltpu.async_copy(key_hbm.at[idx_v], krows, sem1)
            b = pltpu.async_copy(pos_hbm.at[idx_v], prows, sem2)
            a.wait()
            b.wait()
            pltpu.sync_copy(krows, outk_hbm.at[pl.ds(off, _SC_CH)])
            pltpu.sync_copy(prows, outp_hbm.at[pl.ds(off, _SC_CH)])
            return 0

        lax.fori_loop(0, rows_per_w, body, 0)

    return gk(keyT, posT16, idx2d)


def _posmom_body(pg_ref, pq_ref, sum_ref, mom_ref):
    pg = pg_ref[...][:, :8]
    pq = pq_ref[...]
    pq_rep = jnp.broadcast_to(pq[:, None, :], (NT, K, 8)).reshape(ST, 8)
    rel = pq_rep - pg

    @pl.when(pl.program_id(0) == 0)
    def _():
        sum_ref[...] = jnp.zeros_like(sum_ref)
        mom_ref[...] = jnp.zeros_like(mom_ref)

    sum_ref[...] += jnp.sum(rel, axis=0, keepdims=True)
    mom_ref[...] += lax.dot_general(
        rel, rel, (((0,), (0,)), ((), ())),
        preferred_element_type=jnp.float32)


def _posmom(posg, qpos):
    return pl.pallas_call(
        _posmom_body,
        grid=(N // NT,),
        in_specs=[
            pl.BlockSpec((ST, 16), lambda i: (i, 0)),
            pl.BlockSpec((NT, 8), lambda i: (i, 0)),
        ],
        out_specs=[
            pl.BlockSpec((1, 8), lambda i: (0, 0)),
            pl.BlockSpec((8, 8), lambda i: (0, 0)),
        ],
        out_shape=[
            jax.ShapeDtypeStruct((1, 8), jnp.float32),
            jax.ShapeDtypeStruct((8, 8), jnp.float32),
        ],
    )(posg, qpos)


def _compute_s(kg_ref, pg_ref, pq_ref, q_ref, w1f_ref, b1f_ref, w2_ref, b2_ref):
    pg = pg_ref[...][:, :8]
    pq = pq_ref[...]
    pq_rep = jnp.broadcast_to(pq[:, None, :], (NT, K, 8)).reshape(ST, 8)
    rel = pq_rep - pg
    pe1 = jnp.dot(rel, w1f_ref[...], preferred_element_type=jnp.float32)
    pe1 = jnp.maximum(pe1 + b1f_ref[...], 0.0)
    pe = jnp.dot(pe1, w2_ref[...], preferred_element_type=jnp.float32) + b2_ref[...]
    q = q_ref[...]
    q_rep = jnp.broadcast_to(q[:, None, :], (NT, K, DIM)).reshape(ST, DIM)
    s = (q_rep - kg_ref[...]) + pe
    return s, pe


def _smom_body(kg_ref, pg_ref, pq_ref, q_ref, w1f_ref, b1f_ref, w2_ref,
               b2_ref, sum_ref, mom_ref):
    s, _ = _compute_s(kg_ref, pg_ref, pq_ref, q_ref, w1f_ref, b1f_ref,
                      w2_ref, b2_ref)

    @pl.when(pl.program_id(0) == 0)
    def _():
        sum_ref[...] = jnp.zeros_like(sum_ref)
        mom_ref[...] = jnp.zeros_like(mom_ref)

    sum_ref[...] += jnp.sum(s, axis=0, keepdims=True)
    mom_ref[...] += lax.dot_general(
        s, s, (((0,), (0,)), ((), ())),
        preferred_element_type=jnp.float32)


def _smom(keyg, posg, qpos, q, w1f, b1f, w2T, b2):
    return pl.pallas_call(
        _smom_body,
        grid=(N // NT,),
        in_specs=[
            pl.BlockSpec((ST, DIM), lambda i: (i, 0)),
            pl.BlockSpec((ST, 16), lambda i: (i, 0)),
            pl.BlockSpec((NT, 8), lambda i: (i, 0)),
            pl.BlockSpec((NT, DIM), lambda i: (i, 0)),
            pl.BlockSpec((8, DIM), lambda i: (0, 0)),
            pl.BlockSpec((1, DIM), lambda i: (0, 0)),
            pl.BlockSpec((DIM, DIM), lambda i: (0, 0)),
            pl.BlockSpec((1, DIM), lambda i: (0, 0)),
        ],
        out_specs=[
            pl.BlockSpec((1, DIM), lambda i: (0, 0)),
            pl.BlockSpec((DIM, DIM), lambda i: (0, 0)),
        ],
        out_shape=[
            jax.ShapeDtypeStruct((1, DIM), jnp.float32),
            jax.ShapeDtypeStruct((DIM, DIM), jnp.float32),
        ],
    )(keyg, posg, qpos, q, w1f, b1f, w2T, b2)


def _main_body(kg_ref, pg_ref, pq_ref, q_ref, v_ref, xT_ref, w1f_ref,
               b1f_ref, w2_ref, b2_ref, wa1_ref, ba1_ref, wa2_ref, ba2_ref,
               we_ref, be_ref, o_ref):
    s, pe = _compute_s(kg_ref, pg_ref, pq_ref, q_ref, w1f_ref, b1f_ref,
                       w2_ref, b2_ref)
    a1 = jnp.dot(s, wa1_ref[...], preferred_element_type=jnp.float32)
    a1 = jnp.maximum(a1 + ba1_ref[...], 0.0)
    a2 = jnp.dot(a1, wa2_ref[...], preferred_element_type=jnp.float32)
    a2 = (a2 + ba2_ref[...]).reshape(NT, K, DIM)
    mx = jnp.max(a2, axis=1, keepdims=True)
    e = jnp.exp(a2 - mx)
    att = e / jnp.sum(e, axis=1, keepdims=True)
    val = v_ref[...][:, None, :] + pe.reshape(NT, K, DIM)
    agg = jnp.sum(att * val, axis=1)
    y = jnp.dot(agg, we_ref[...], preferred_element_type=jnp.float32)
    o_ref[...] = y + be_ref[...] + xT_ref[...]


def _main(keyg, posg, qpos, q, v, xT, w1f, b1f, w2T, b2, wa1f, ba1f, wa2T,
          ba2, weT, be):
    return pl.pallas_call(
        _main_body,
        grid=(N // NT,),
        in_specs=[
            pl.BlockSpec((ST, DIM), lambda i: (i, 0)),
            pl.BlockSpec((ST, 16), lambda i: (i, 0)),
            pl.BlockSpec((NT, 8), lambda i: (i, 0)),
            pl.BlockSpec((NT, DIM), lambda i: (i, 0)),
            pl.BlockSpec((NT, DIM), lambda i: (i, 0)),
            pl.BlockSpec((NT, CIN), lambda i: (i, 0)),
            pl.BlockSpec((8, DIM), lambda i: (0, 0)),
            pl.BlockSpec((1, DIM), lambda i: (0, 0)),
            pl.BlockSpec((DIM, DIM), lambda i: (0, 0)),
            pl.BlockSpec((1, DIM), lambda i: (0, 0)),
            pl.BlockSpec((DIM, HID), lambda i: (0, 0)),
            pl.BlockSpec((1, HID), lambda i: (0, 0)),
            pl.BlockSpec((HID, DIM), lambda i: (0, 0)),
            pl.BlockSpec((1, DIM), lambda i: (0, 0)),
            pl.BlockSpec((DIM, CIN), lambda i: (0, 0)),
            pl.BlockSpec((1, CIN), lambda i: (0, 0)),
        ],
        out_specs=pl.BlockSpec((NT, CIN), lambda i: (i, 0)),
        out_shape=jax.ShapeDtypeStruct((N, CIN), jnp.float32),
    )(keyg, posg, qpos, q, v, xT, w1f, b1f, w2T, b2, wa1f, ba1f, wa2T,
      ba2, weT, be)


def _bn_fold(w, b, mu, mom, g, be, n):
    mu = mu / n
    mom = mom / n
    mean = w @ mu + b
    ex2 = jnp.sum((w @ mom) * w, axis=1) + 2.0 * b * (w @ mu) + b * b
    var = jnp.maximum(ex2 - mean * mean, 0.0)
    scale = g / jnp.sqrt(var + 1e-5)
    shift = be - mean * scale
    wf = w * scale[:, None]
    bf = b * scale + shift
    return wf.T, bf[None, :]


def kernel(x, pos, w_start, b_start, w_key, b_key, w_query, b_query,
           w_value, b_value, w_pos1, b_pos1, g_pos, be_pos, w_pos2, b_pos2,
           w_attn1, b_attn1, g_attn, be_attn, w_attn2, b_attn2, w_end,
           b_end):
    f32 = jnp.float32
    xT = jnp.transpose(x[0])
    posT = jnp.transpose(pos[0])

    wqkv = jnp.concatenate([w_query, w_key, w_value], axis=0)
    bqkv = jnp.concatenate([b_query, b_key, b_value], axis=0)
    qkv = _qkv(xT, w_start.T, wqkv.T) + (b_start @ wqkv.T + bqkv)[None, :]
    q = qkv[:, :DIM]
    keyT = qkv[:, DIM:2 * DIM]
    v = qkv[:, 2 * DIM:]

    qpos = jnp.zeros((NPAD, 8), f32).at[:N, :3].set(posT)
    pos8 = qpos.T
    idx = _knn(qpos, pos8)[:N]

    posT16 = jnp.zeros((N, 16), f32).at[:, :3].set(posT)
    idx_flat = idx.reshape(-1)
    idx2d = jnp.zeros((NSAMP_PAD,), jnp.int32).at[:NSAMP].set(idx_flat)
    idx2d = idx2d.reshape(NSAMP_PAD // _SC_CH, _SC_CH)
    if _SC_USE:
        keyg, posg = _sc_gather(keyT, posT16, idx2d)
    else:
        keyg = keyT[idx_flat]
        posg = posT16[idx_flat]

    qpos_n = qpos[:N]

    psum, pmom = _posmom(posg, qpos_n)
    w1p = jnp.zeros((DIM, 8), f32).at[:, :3].set(w_pos1)
    w1f, b1f = _bn_fold(w1p, b_pos1, psum[0], pmom, g_pos, be_pos,
                        float(NSAMP))

    ssum, smom = _smom(keyg, posg, qpos_n, q, w1f, b1f, w_pos2.T,
                       b_pos2[None, :])
    wa1f, ba1f = _bn_fold(w_attn1, b_attn1, ssum[0], smom, g_attn, be_attn,
                          float(NSAMP))

    out = _main(keyg, posg, qpos_n, q, v, xT, w1f, b1f, w_pos2.T,
                b_pos2[None, :], wa1f, ba1f, w_attn2.T, b_attn2[None, :],
                w_end.T, b_end[None, :])
    return jnp.transpose(out)[None]


# speedup vs baseline: 110.6304x; 1.1801x over previous
import functools

import jax
import jax.numpy as jnp
from jax import lax
from jax.experimental import pallas as pl
from jax.experimental.pallas import tpu as pltpu
from jax.experimental.pallas import tpu_sc as plsc

N = 10000
K = 16
NPAD = 10112
DIM = 64
CIN = 128
HID = 256
QTILE = 128
NT = 200
ST = NT * K
NSAMP = N * K
NSAMP_PAD = 163840
BIGF = 3.0e38


def _qkv_body(x_ref, ws_ref, wqkv_ref, o_ref):
    xf = jnp.dot(x_ref[...], ws_ref[...], preferred_element_type=jnp.float32)
    o_ref[...] = jnp.dot(xf, wqkv_ref[...], preferred_element_type=jnp.float32)


def _qkv(xT, wsT, wqkvT):
    tn = 1000
    return pl.pallas_call(
        _qkv_body,
        grid=(N // tn,),
        in_specs=[
            pl.BlockSpec((tn, CIN), lambda i: (i, 0)),
            pl.BlockSpec((CIN, DIM), lambda i: (0, 0)),
            pl.BlockSpec((DIM, 3 * DIM), lambda i: (0, 0)),
        ],
        out_specs=pl.BlockSpec((tn, 3 * DIM), lambda i: (i, 0)),
        out_shape=jax.ShapeDtypeStruct((N, 3 * DIM), jnp.float32),
    )(xT, wsT, wqkvT)


NCHUNK = NPAD // 128
CTOP = 4


def _knn_body(q_ref, p_ref, o_ref):
    Q = q_ref[...]
    P = p_ref[...]
    S = jnp.sum(P * P, axis=0, keepdims=True)
    D = S - 2.0 * jnp.dot(Q, P, preferred_element_type=jnp.float32)
    colid = lax.broadcasted_iota(jnp.int32, (QTILE, NPAD), 1)
    D = jnp.where(colid >= N, BIGF, D)

    D3 = D.reshape(QTILE, NCHUNK, 128)
    lane_f = lax.broadcasted_iota(
        jnp.int32, (QTILE, NCHUNK, 128), 2).astype(jnp.float32)
    chunk_base = lax.broadcasted_iota(
        jnp.int32, (QTILE, NCHUNK), 1).astype(jnp.float32) * 128.0
    prev_c = jnp.full((QTILE, NCHUNK, 1), -BIGF, jnp.float32)
    vals, idxs = [], []
    for _ in range(CTOP):
        m = jnp.min(jnp.where(D3 > prev_c, D3, BIGF), axis=2, keepdims=True)
        lane = jnp.min(jnp.where(D3 == m, lane_f, BIGF), axis=2)
        vals.append(m[:, :, 0])
        idxs.append(chunk_base + lane)
        prev_c = m
    V = jnp.concatenate(vals, axis=1)
    I = jnp.concatenate(idxs, axis=1)

    prev = jnp.full((QTILE, 1), -BIGF, jnp.float32)
    cols = []
    for _ in range(K):
        m = jnp.min(jnp.where(V > prev, V, BIGF), axis=1, keepdims=True)
        sel = jnp.min(jnp.where(V == m, I, BIGF), axis=1, keepdims=True)
        cols.append(sel)
        prev = m
    t = prev
    o_ref[...] = jnp.concatenate(cols, axis=1).astype(jnp.int32)

    cnt = jnp.sum(jnp.where(D3 <= t[:, :, None], 1, 0).astype(jnp.int32),
                  axis=2)
    bad = jnp.max(cnt)

    @pl.when(bad > CTOP)
    def _():
        colid_f = colid.astype(jnp.float32)
        prev2 = jnp.full((QTILE, 1), -BIGF, jnp.float32)
        cols2 = []
        for _ in range(K):
            m2 = jnp.min(jnp.where(D > prev2, D, BIGF), axis=1,
                         keepdims=True)
            sel2 = jnp.min(jnp.where(D == m2, colid_f, BIGF), axis=1,
                           keepdims=True)
            cols2.append(sel2)
            prev2 = m2
        o_ref[...] = jnp.concatenate(cols2, axis=1).astype(jnp.int32)


def _knn(qpos, pos8):
    return pl.pallas_call(
        _knn_body,
        grid=(NPAD // QTILE,),
        in_specs=[
            pl.BlockSpec((QTILE, 8), lambda i: (i, 0)),
            pl.BlockSpec((8, NPAD), lambda i: (0, 0)),
        ],
        out_specs=pl.BlockSpec((QTILE, K), lambda i: (i, 0)),
        out_shape=jax.ShapeDtypeStruct((NPAD, K), jnp.int32),
    )(qpos, pos8)


_SC_CH = 128
_SC_USE = True


def _sc_gather(keyT, posT16, idx2d):
    info = plsc.get_sparse_core_info()
    nw = info.num_cores * info.num_subcores
    rows_per_w = idx2d.shape[0] // nw
    mesh = plsc.VectorSubcoreMesh(core_axis_name="c", subcore_axis_name="s")

    @functools.partial(
        pl.kernel,
        mesh=mesh,
        compiler_params=pltpu.CompilerParams(use_tc_tiling_on_sc=False),
        out_type=[
            jax.ShapeDtypeStruct((NSAMP_PAD, DIM), jnp.float32),
            jax.ShapeDtypeStruct((NSAMP_PAD, 16), jnp.float32),
        ],
        scratch_types=[
            pltpu.VMEM((_SC_CH,), jnp.int32),
            pltpu.VMEM((_SC_CH, DIM), jnp.float32),
            pltpu.VMEM((_SC_CH, 16), jnp.float32),
            pltpu.SemaphoreType.DMA,
            pltpu.SemaphoreType.DMA,
        ],
    )
    def gk(key_hbm, pos_hbm, idx_hbm, outk_hbm, outp_hbm,
           idx_v, krows, prows, sem1, sem2):
        wid = lax.axis_index("s") * info.num_cores + lax.axis_index("c")
        row0 = wid * rows_per_w

        def body(i, _):
            row = row0 + i
            off = row * _SC_CH
            pltpu.sync_copy(idx_hbm.at[row], idx_v)
            a = pltpu.async_copy(key_hbm.at[idx_v], krows, sem1)
            b = pltpu.async_copy(pos_hbm.at[idx_v], prows, sem2)
            a.wait()
            b.wait()
            pltpu.sync_copy(krows, outk_hbm.at[pl.ds(off, _SC_CH)])
            pltpu.sync_copy(prows, outp_hbm.at[pl.ds(off, _SC_CH)])
            return 0

        lax.fori_loop(0, rows_per_w, body, 0)

    return gk(keyT, posT16, idx2d)


def _posmom_body(pg_ref, pq_ref, sum_ref, mom_ref):
    pg = pg_ref[...][:, :8]
    pq = pq_ref[...]
    pq_rep = jnp.broadcast_to(pq[:, None, :], (NT, K, 8)).reshape(ST, 8)
    rel = pq_rep - pg

    @pl.when(pl.program_id(0) == 0)
    def _():
        sum_ref[...] = jnp.zeros_like(sum_ref)
        mom_ref[...] = jnp.zeros_like(mom_ref)

    sum_ref[...] += jnp.sum(rel, axis=0, keepdims=True)
    mom_ref[...] += lax.dot_general(
        rel, rel, (((0,), (0,)), ((), ())),
        preferred_element_type=jnp.float32)


def _posmom(posg, qpos):
    return pl.pallas_call(
        _posmom_body,
        grid=(N // NT,),
        in_specs=[
            pl.BlockSpec((ST, 16), lambda i: (i, 0)),
            pl.BlockSpec((NT, 8), lambda i: (i, 0)),
        ],
        out_specs=[
            pl.BlockSpec((1, 8), lambda i: (0, 0)),
            pl.BlockSpec((8, 8), lambda i: (0, 0)),
        ],
        out_shape=[
            jax.ShapeDtypeStruct((1, 8), jnp.float32),
            jax.ShapeDtypeStruct((8, 8), jnp.float32),
        ],
    )(posg, qpos)


def _compute_s(kg_ref, pg_ref, pq_ref, q_ref, w1f_ref, b1f_ref, w2_ref, b2_ref):
    pg = pg_ref[...][:, :8]
    pq = pq_ref[...]
    pq_rep = jnp.broadcast_to(pq[:, None, :], (NT, K, 8)).reshape(ST, 8)
    rel = pq_rep - pg
    pe1 = jnp.dot(rel, w1f_ref[...], preferred_element_type=jnp.float32)
    pe1 = jnp.maximum(pe1 + b1f_ref[...], 0.0)
    pe = jnp.dot(pe1, w2_ref[...], preferred_element_type=jnp.float32) + b2_ref[...]
    q = q_ref[...]
    q_rep = jnp.broadcast_to(q[:, None, :], (NT, K, DIM)).reshape(ST, DIM)
    s = (q_rep - kg_ref[...]) + pe
    return s, pe


def _smom_body(kg_ref, pg_ref, pq_ref, q_ref, w1f_ref, b1f_ref, w2_ref,
               b2_ref, sum_ref, mom_ref):
    s, _ = _compute_s(kg_ref, pg_ref, pq_ref, q_ref, w1f_ref, b1f_ref,
                      w2_ref, b2_ref)

    @pl.when(pl.program_id(0) == 0)
    def _():
        sum_ref[...] = jnp.zeros_like(sum_ref)
        mom_ref[...] = jnp.zeros_like(mom_ref)

    sum_ref[...] += jnp.sum(s, axis=0, keepdims=True)
    mom_ref[...] += lax.dot_general(
        s, s, (((0,), (0,)), ((), ())),
        preferred_element_type=jnp.float32)


def _smom(keyg, posg, qpos, q, w1f, b1f, w2T, b2):
    return pl.pallas_call(
        _smom_body,
        grid=(N // NT,),
        in_specs=[
            pl.BlockSpec((ST, DIM), lambda i: (i, 0)),
            pl.BlockSpec((ST, 16), lambda i: (i, 0)),
            pl.BlockSpec((NT, 8), lambda i: (i, 0)),
            pl.BlockSpec((NT, DIM), lambda i: (i, 0)),
            pl.BlockSpec((8, DIM), lambda i: (0, 0)),
            pl.BlockSpec((1, DIM), lambda i: (0, 0)),
            pl.BlockSpec((DIM, DIM), lambda i: (0, 0)),
            pl.BlockSpec((1, DIM), lambda i: (0, 0)),
        ],
        out_specs=[
            pl.BlockSpec((1, DIM), lambda i: (0, 0)),
            pl.BlockSpec((DIM, DIM), lambda i: (0, 0)),
        ],
        out_shape=[
            jax.ShapeDtypeStruct((1, DIM), jnp.float32),
            jax.ShapeDtypeStruct((DIM, DIM), jnp.float32),
        ],
    )(keyg, posg, qpos, q, w1f, b1f, w2T, b2)


def _main_body(kg_ref, pg_ref, pq_ref, q_ref, v_ref, xT_ref, w1f_ref,
               b1f_ref, w2_ref, b2_ref, wa1_ref, ba1_ref, wa2_ref, ba2_ref,
               we_ref, be_ref, o_ref):
    s, pe = _compute_s(kg_ref, pg_ref, pq_ref, q_ref, w1f_ref, b1f_ref,
                       w2_ref, b2_ref)
    a1 = jnp.dot(s, wa1_ref[...], preferred_element_type=jnp.float32)
    a1 = jnp.maximum(a1 + ba1_ref[...], 0.0)
    a2 = jnp.dot(a1, wa2_ref[...], preferred_element_type=jnp.float32)
    a2 = (a2 + ba2_ref[...]).reshape(NT, K, DIM)
    mx = jnp.max(a2, axis=1, keepdims=True)
    e = jnp.exp(a2 - mx)
    att = e / jnp.sum(e, axis=1, keepdims=True)
    val = v_ref[...][:, None, :] + pe.reshape(NT, K, DIM)
    agg = jnp.sum(att * val, axis=1)
    y = jnp.dot(agg, we_ref[...], preferred_element_type=jnp.float32)
    o_ref[...] = y + be_ref[...] + xT_ref[...]


def _main(keyg, posg, qpos, q, v, xT, w1f, b1f, w2T, b2, wa1f, ba1f, wa2T,
          ba2, weT, be):
    return pl.pallas_call(
        _main_body,
        grid=(N // NT,),
        in_specs=[
            pl.BlockSpec((ST, DIM), lambda i: (i, 0)),
            pl.BlockSpec((ST, 16), lambda i: (i, 0)),
            pl.BlockSpec((NT, 8), lambda i: (i, 0)),
            pl.BlockSpec((NT, DIM), lambda i: (i, 0)),
            pl.BlockSpec((NT, DIM), lambda i: (i, 0)),
            pl.BlockSpec((NT, CIN), lambda i: (i, 0)),
            pl.BlockSpec((8, DIM), lambda i: (0, 0)),
            pl.BlockSpec((1, DIM), lambda i: (0, 0)),
            pl.BlockSpec((DIM, DIM), lambda i: (0, 0)),
            pl.BlockSpec((1, DIM), lambda i: (0, 0)),
            pl.BlockSpec((DIM, HID), lambda i: (0, 0)),
            pl.BlockSpec((1, HID), lambda i: (0, 0)),
            pl.BlockSpec((HID, DIM), lambda i: (0, 0)),
            pl.BlockSpec((1, DIM), lambda i: (0, 0)),
            pl.BlockSpec((DIM, CIN), lambda i: (0, 0)),
            pl.BlockSpec((1, CIN), lambda i: (0, 0)),
        ],
        out_specs=pl.BlockSpec((NT, CIN), lambda i: (i, 0)),
        out_shape=jax.ShapeDtypeStruct((N, CIN), jnp.float32),
    )(keyg, posg, qpos, q, v, xT, w1f, b1f, w2T, b2, wa1f, ba1f, wa2T,
      ba2, weT, be)


def _bn_fold(w, b, mu, mom, g, be, n):
    mu = mu / n
    mom = mom / n
    mean = w @ mu + b
    ex2 = jnp.sum((w @ mom) * w, axis=1) + 2.0 * b * (w @ mu) + b * b
    var = jnp.maximum(ex2 - mean * mean, 0.0)
    scale = g / jnp.sqrt(var + 1e-5)
    shift = be - mean * scale
    wf = w * scale[:, None]
    bf = b * scale + shift
    return wf.T, bf[None, :]


def kernel(x, pos, w_start, b_start, w_key, b_key, w_query, b_query,
           w_value, b_value, w_pos1, b_pos1, g_pos, be_pos, w_pos2, b_pos2,
           w_attn1, b_attn1, g_attn, be_attn, w_attn2, b_attn2, w_end,
           b_end):
    f32 = jnp.float32
    xT = jnp.transpose(x[0])
    posT = jnp.transpose(pos[0])

    wqkv = jnp.concatenate([w_query, w_key, w_value], axis=0)
    bqkv = jnp.concatenate([b_query, b_key, b_value], axis=0)
    qkv = _qkv(xT, w_start.T, wqkv.T) + (b_start @ wqkv.T + bqkv)[None, :]
    q = qkv[:, :DIM]
    keyT = qkv[:, DIM:2 * DIM]
    v = qkv[:, 2 * DIM:]

    qpos = jnp.zeros((NPAD, 8), f32).at[:N, :3].set(posT)
    pos8 = qpos.T
    idx = _knn(qpos, pos8)[:N]

    posT16 = jnp.zeros((N, 16), f32).at[:, :3].set(posT)
    idx_flat = idx.reshape(-1)
    idx2d = jnp.zeros((NSAMP_PAD,), jnp.int32).at[:NSAMP].set(idx_flat)
    idx2d = idx2d.reshape(NSAMP_PAD // _SC_CH, _SC_CH)
    if _SC_USE:
        keyg, posg = _sc_gather(keyT, posT16, idx2d)
    else:
        keyg = keyT[idx_flat]
        posg = posT16[idx_flat]

    qpos_n = qpos[:N]

    psum, pmom = _posmom(posg, qpos_n)
    w1p = jnp.zeros((DIM, 8), f32).at[:, :3].set(w_pos1)
    w1f, b1f = _bn_fold(w1p, b_pos1, psum[0], pmom, g_pos, be_pos,
                        float(NSAMP))

    ssum, smom = _smom(keyg, posg, qpos_n, q, w1f, b1f, w_pos2.T,
                       b_pos2[None, :])
    wa1f, ba1f = _bn_fold(w_attn1, b_attn1, ssum[0], smom, g_attn, be_attn,
                          float(NSAMP))

    out = _main(keyg, posg, qpos_n, q, v, xT, w1f, b1f, w_pos2.T,
                b_pos2[None, :], wa1f, ba1f, w_attn2.T, b_attn2[None, :],
                w_end.T, b_end[None, :])
    return jnp.transpose(out)[None]
